# baseline (speedup 1.0000x reference)
import functools

import jax
import jax.numpy as jnp
from jax import lax
from jax.experimental import pallas as pl
from jax.experimental.pallas import tpu as pltpu
from jax.experimental.pallas import tpu_sc as plsc

NUM_SELECTS = 8
BLOCK_T = 1024

_T = 8192
_E = 64
_NW = 32
_TPW = _T // _NW


def _logits_body(x_ref, w_ref, out_ref):
    out_ref[...] = jax.lax.dot_general(
        x_ref[...], w_ref[...],
        dimension_numbers=(((1,), (1,)), ((), ())),
        preferred_element_type=jnp.float32,
    )


@jax.jit
def _tc_logits(x, w_route):
    t, d = x.shape
    e = w_route.shape[0]
    return pl.pallas_call(
        _logits_body,
        grid=(t // BLOCK_T,),
        in_specs=[
            pl.BlockSpec((BLOCK_T, d), lambda i: (i, 0)),
            pl.BlockSpec((e, d), lambda i: (0, 0)),
        ],
        out_specs=pl.BlockSpec((BLOCK_T, e), lambda i: (i, 0)),
        out_shape=jax.ShapeDtypeStruct((t, e), jnp.float32),
        compiler_params=pltpu.CompilerParams(
            dimension_semantics=("parallel",),
        ),
    )(x, w_route)


@functools.partial(
    pl.kernel,
    mesh=plsc.VectorSubcoreMesh(core_axis_name="c", subcore_axis_name="s"),
    out_type=[
        jax.ShapeDtypeStruct((_T, 16), jnp.float32),
        jax.ShapeDtypeStruct((_T, 16), jnp.int32),
    ],
    scratch_types=[
        pltpu.VMEM((_TPW, _E), jnp.float32),
        pltpu.VMEM((_TPW, 16), jnp.float32),
        pltpu.VMEM((_TPW, 16), jnp.int32),
    ],
    compiler_params=pltpu.CompilerParams(needs_layout_passes=False),
)
def _sc_topk(logits_hbm, gate_hbm, idx_hbm, lg_v, g_v, i_v):
    wid = lax.axis_index("s") * 2 + lax.axis_index("c")
    base = wid * _TPW
    pltpu.sync_copy(logits_hbm.at[pl.ds(base, _TPW)], lg_v)

    lane = lax.iota(jnp.int32, 16)

    def merge(ak, av, bk, bv):
        rbk = lax.rev(bk, (0,))
        rbv = lax.rev(bv, (0,))
        eq = ak == rbk
        pref = jnp.where(eq, av < rbv, ak > rbk)
        hk = jnp.where(pref, ak, rbk)
        hv = jnp.where(pref, av, rbv)
        return plsc.sort_key_val(hk, hv, descending=True)

    def body(t, carry):
        ks = []
        vs = []
        for c in range(4):
            k, v = plsc.sort_key_val(
                lg_v[t, c * 16:(c + 1) * 16], lane + (c * 16),
                descending=True)
            ks.append(k)
            vs.append(v)
        k01, v01 = merge(ks[0], vs[0], ks[1], vs[1])
        k23, v23 = merge(ks[2], vs[2], ks[3], vs[3])
        fk, fv = merge(k01, v01, k23, v23)
        m = jnp.max(fk)
        ex = jnp.where(lane < NUM_SELECTS, jnp.exp(fk - m), 0.0)
        g_v[t, :] = ex / jnp.sum(ex)
        i_v[t, :] = fv
        return carry

    lax.fori_loop(0, _TPW, body, 0)
    pltpu.sync_copy(g_v, gate_hbm.at[pl.ds(base, _TPW)])
    pltpu.sync_copy(i_v, idx_hbm.at[pl.ds(base, _TPW)])


def kernel(x, W_route, W_noise):
    logits = _tc_logits(x, W_route)
    wide_g, wide_i = _sc_topk(logits)
    return wide_g[:, :NUM_SELECTS], wide_i[:, :NUM_SELECTS]


# speedup vs baseline: 1.4246x; 1.4246x over previous
import jax
import jax.numpy as jnp
from jax.experimental import pallas as pl
from jax.experimental.pallas import tpu as pltpu

NUM_SELECTS = 8
BLOCK_T = 1024
SUB_T = 128


def _topk_softmax_t(logits_t, gate_ref, idx_ref, row0):
    e, st = logits_t.shape
    row = jax.lax.broadcasted_iota(jnp.int32, (e, st), 0).astype(jnp.float32)
    neg = jnp.finfo(jnp.float32).min
    vals = logits_t
    top_v = []
    top_i = []
    for k in range(NUM_SELECTS):
        m = jnp.max(vals, axis=0, keepdims=True)
        idx = jnp.min(jnp.where(vals == m, row, float(e)), axis=0,
                      keepdims=True)
        top_v.append(m)
        top_i.append(idx)
        if k + 1 < NUM_SELECTS:
            vals = jnp.where(row == idx, neg, vals)
    v = jnp.concatenate(top_v, axis=0)
    i = jnp.concatenate(top_i, axis=0)
    ex = jnp.exp(v - v[0:1, :])
    g = ex / jnp.sum(ex, axis=0, keepdims=True)
    gate_ref[pl.ds(row0, st), :] = g.T
    idx_ref[pl.ds(row0, st), :] = i.T.astype(jnp.int32)


def _router_body(x_ref, w_ref, gate_ref, idx_ref):
    bt = x_ref.shape[0]
    w = w_ref[...]
    for s in range(bt // SUB_T):
        logits_t = jax.lax.dot_general(
            w, x_ref[s * SUB_T:(s + 1) * SUB_T, :],
            dimension_numbers=(((1,), (1,)), ((), ())),
            preferred_element_type=jnp.float32,
        )
        _topk_softmax_t(logits_t, gate_ref, idx_ref, s * SUB_T)


@jax.jit
def _router(x, w_route):
    t, d = x.shape
    e = w_route.shape[0]
    grid = (t // BLOCK_T,)
    return pl.pallas_call(
        _router_body,
        grid=grid,
        in_specs=[
            pl.BlockSpec((BLOCK_T, d), lambda i: (i, 0)),
            pl.BlockSpec((e, d), lambda i: (0, 0)),
        ],
        out_specs=[
            pl.BlockSpec((BLOCK_T, NUM_SELECTS), lambda i: (i, 0)),
            pl.BlockSpec((BLOCK_T, NUM_SELECTS), lambda i: (i, 0)),
        ],
        out_shape=[
            jax.ShapeDtypeStruct((t, NUM_SELECTS), jnp.float32),
            jax.ShapeDtypeStruct((t, NUM_SELECTS), jnp.int32),
        ],
        compiler_params=pltpu.CompilerParams(
            dimension_semantics=("parallel",),
        ),
    )(x, w_route)


def kernel(x, W_route, W_noise):
    gates, idx = _router(x, W_route)
    return gates, idx
